# baseline (speedup 1.0000x reference)
import functools

import jax
import jax.numpy as jnp
from jax import lax
from jax.experimental import pallas as pl
from jax.experimental.pallas import tpu as pltpu
from jax.experimental.pallas import tpu_sc as plsc

_D = 1000
_N = 1024 * 50
_NC, _NS = 2, 16
_NW = _NC * _NS
_RPW = _N // _NW
_WIN = 16


def _sc_gather(x_flat, prob):
  mesh = plsc.VectorSubcoreMesh(core_axis_name="c", subcore_axis_name="s")

  @functools.partial(
      pl.kernel,
      out_type=jax.ShapeDtypeStruct((_N, _D), jnp.float32),
      mesh=mesh,
      scratch_types=[
          pltpu.VMEM((_RPW,), jnp.int32),
          pltpu.SemaphoreType.DMA,
      ],
  )
  def body(idx_hbm, table_hbm, out_hbm, idx_v, sem):
    wid = lax.axis_index("s") * _NC + lax.axis_index("c")
    base = wid * _RPW
    pltpu.sync_copy(idx_hbm.at[pl.ds(base, _RPW)], idx_v)

    def issue_group(g):
      off = pl.multiple_of(g * 16, 16)
      v = idx_v[pl.ds(off, 16)]
      for k in range(16):
        pltpu.make_async_copy(
            table_hbm.at[v[k]], out_hbm.at[base + off + k], sem).start()

    def drain_one():
      pltpu.make_async_copy(table_hbm.at[0], out_hbm.at[base], sem).wait()

    issue_group(0)

    def step(g, carry):
      issue_group(g)
      for _ in range(16):
        drain_one()
      return carry

    lax.fori_loop(1, _RPW // 16, step, 0)
    for _ in range(16):
      drain_one()

  return body(x_flat, prob)


def kernel(x, prob):
  x_flat = x.reshape(-1)
  out = _sc_gather(x_flat, prob)
  return out.reshape(x.shape[0], x.shape[1], _D)


# speedup vs baseline: 9.7310x; 9.7310x over previous
import functools

import jax
import jax.numpy as jnp
from jax import lax
from jax.experimental import pallas as pl
from jax.experimental.pallas import tpu as pltpu
from jax.experimental.pallas import tpu_sc as plsc

_D = 1000
_B, _T = 1024, 50
_NC, _NS = 2, 16
_NW = _NC * _NS
_BPW = _B // _NW


def _sc_gather(x, prob):
  mesh = plsc.VectorSubcoreMesh(core_axis_name="c", subcore_axis_name="s")

  @functools.partial(
      pl.kernel,
      out_type=jax.ShapeDtypeStruct((_B, _T, _D), jnp.float32),
      mesh=mesh,
      scratch_types=[
          pltpu.VMEM((_BPW, _T), jnp.int32),
          pltpu.VMEM((_T, _D), jnp.float32),
          pltpu.VMEM((_T, _D), jnp.float32),
          pltpu.SemaphoreType.DMA,
          pltpu.SemaphoreType.DMA,
          pltpu.SemaphoreType.DMA,
          pltpu.SemaphoreType.DMA,
      ],
      compiler_params=pltpu.CompilerParams(use_tc_tiling_on_sc=False),
  )
  def body(idx_hbm, table_hbm, out_hbm, idx_v, rows0, rows1, g0, g1, s0, s1):
    wid = lax.axis_index("s") * _NC + lax.axis_index("c")
    b0 = wid * _BPW
    pltpu.sync_copy(idx_hbm.at[pl.ds(b0, _BPW)], idx_v)

    bufs = (rows0, rows1)
    gsems = (g0, g1)
    ssems = (s0, s1)

    def gather(c, p):
      return pltpu.make_async_copy(
          table_hbm.at[idx_v.at[c]], bufs[p], gsems[p])

    def scatter(c, p):
      return pltpu.make_async_copy(bufs[p], out_hbm.at[b0 + c], ssems[p])

    gather(0, 0).start()
    gather(1, 1).start()

    def step(jj, carry):
      c0 = 2 * jj
      gather(c0, 0).wait()
      scatter(c0, 0).start()
      gather(c0 + 1, 1).wait()
      scatter(c0 + 1, 1).start()
      scatter(c0, 0).wait()
      gather(c0 + 2, 0).start()
      scatter(c0 + 1, 1).wait()
      gather(c0 + 3, 1).start()
      return carry

    lax.fori_loop(0, _BPW // 2 - 1, step, 0)

    cl = _BPW - 2
    gather(cl, 0).wait()
    scatter(cl, 0).start()
    gather(cl + 1, 1).wait()
    scatter(cl + 1, 1).start()
    scatter(cl, 0).wait()
    scatter(cl + 1, 1).wait()

  return body(x, prob)


def kernel(x, prob):
  return _sc_gather(x, prob)
